# baseline (speedup 1.0000x reference)
import functools

import jax
import jax.numpy as jnp
from jax import lax
from jax.experimental import pallas as pl
from jax.experimental.pallas import tpu as pltpu
from jax.experimental.pallas import tpu_sc as plsc

B, L, D_CLN, D_EMB = 4, 256, 128, 64
ROWS = B * L
N = ROWS * L
NC, NS = 2, 16
NW = NC * NS
ROWS_PW = ROWS // NW
POS_PW = ROWS_PW * L
GCH = 128
CH = 256
NCH = POS_PW // CH
RB = 64
REP = 8


def _sc_body(dt_hbm, rt_hbm, dist_hbm, mask_hbm, out_hbm,
             ct_v, dt_v, rt_v, ctmp_v, dist_v, mask_v, idx_v, stage_v, sem):
    wid = lax.axis_index("s") * NC + lax.axis_index("c")
    base = wid * POS_PW

    sid = lax.axis_index("s")

    @pl.when(sid == 0)
    def _():
        pltpu.sync_copy(dt_hbm, dt_v)
        pltpu.sync_copy(rt_hbm, rt_v)
        for c in range(60):
            for k in range(D_EMB // 16):
                ctmp_v[c, pl.ds(k * 16, 16)] = dt_v[c // 3, pl.ds(k * 16, 16)]
                ctmp_v[c, pl.ds(D_EMB + k * 16, 16)] = (
                    rt_v[c % 3, pl.ds(k * 16, 16)])
        for rep in range(REP):
            pltpu.sync_copy(ctmp_v, ct_v.at[pl.ds(rep * 64, 64)])

    pltpu.sync_copy(dist_hbm.at[pl.ds(base, POS_PW)], dist_v)
    pltpu.sync_copy(mask_hbm.at[pl.ds(base, POS_PW)], mask_v)

    rep_off = jnp.full((16,), lax.rem(sid, REP) * 64, jnp.int32)

    def idx_row(t, _):
        g = wid * ROWS_PW + t
        i = lax.rem(g, L)
        iv = jnp.full((16,), i, jnp.int32)
        for jj in range(L // 16):
            o = t * L + jj * 16
            d = dist_v[pl.ds(o, 16)]
            m = mask_v[pl.ds(o, 16)]
            j = jj * 16 + lax.iota(jnp.int32, 16)
            ge = jnp.where(j >= iv, jnp.int32(1), jnp.int32(0))
            idx_v[pl.ds(o, 16)] = d * 3 + m * (1 + ge) + rep_off
        return _

    lax.fori_loop(0, ROWS_PW, idx_row, None)
    plsc.subcore_barrier()

    stages = (stage_v.at[0], stage_v.at[1])
    gsems = (sem.at[0], sem.at[1])
    wsems = (sem.at[2], sem.at[3])
    gdesc = [None, None]
    wdesc = [None, None]
    for c in range(NCH):
        b = c % 2
        if c >= 2:
            wdesc[b].wait()
        gdesc[b] = [
            pltpu.async_copy(
                ct_v.at[idx_v.at[pl.ds(c * CH + h * GCH, GCH)]],
                stages[b].at[pl.ds(h * GCH, GCH)], gsems[b])
            for h in range(CH // GCH)
        ]
        if c >= 1:
            p = 1 - b
            for dsc in gdesc[p]:
                dsc.wait()
            wdesc[p] = pltpu.async_copy(
                stages[p],
                out_hbm.at[pl.ds(base + (c - 1) * CH, CH),
                           pl.ds(D_CLN, 2 * D_EMB)], wsems[p])
    last = (NCH - 1) % 2
    for dsc in gdesc[last]:
        dsc.wait()
    wdesc[last] = pltpu.async_copy(
        stages[last],
        out_hbm.at[pl.ds(base + (NCH - 1) * CH, CH),
                   pl.ds(D_CLN, 2 * D_EMB)], wsems[last])
    wdesc[1 - last].wait()
    wdesc[last].wait()


@functools.partial(
    pl.kernel,
    out_type=jax.ShapeDtypeStruct((N, 2 * D_CLN), jnp.float32),
    mesh=plsc.VectorSubcoreMesh(core_axis_name="c", subcore_axis_name="s"),
    scratch_types=[
        pltpu.VMEM_SHARED((64 * REP, 2 * D_EMB), jnp.float32),
        pltpu.VMEM((20, D_EMB), jnp.float32),
        pltpu.VMEM((3, D_EMB), jnp.float32),
        pltpu.VMEM((64, 2 * D_EMB), jnp.float32),
        pltpu.VMEM((POS_PW,), jnp.int32),
        pltpu.VMEM((POS_PW,), jnp.int32),
        pltpu.VMEM((POS_PW,), jnp.int32),
        pltpu.VMEM((2, CH, 2 * D_EMB), jnp.float32),
        pltpu.SemaphoreType.DMA((4,)),
    ],
)
def _sc_fill(dt_hbm, rt_hbm, dist_hbm, mask_hbm, out_hbm,
             ct_v, dt_v, rt_v, ctmp_v, dist_v, mask_v, idx_v, stage_v, sem):
    _sc_body(dt_hbm, rt_hbm, dist_hbm, mask_hbm, out_hbm,
             ct_v, dt_v, rt_v, ctmp_v, dist_v, mask_v, idx_v, stage_v, sem)


def _copy_body(prev_ref, cln_ref, out_ref):
    out_ref[...] = cln_ref[...]


def _fill_cln(sc_out, cln2):
    return pl.pallas_call(
        _copy_body,
        grid=(ROWS // RB,),
        in_specs=[
            pl.BlockSpec(memory_space=pltpu.MemorySpace.HBM),
            pl.BlockSpec((RB, L, D_CLN), lambda r: (r, 0, 0)),
        ],
        out_specs=pl.BlockSpec((RB, L, D_CLN), lambda r: (r, 0, 0)),
        out_shape=jax.ShapeDtypeStruct((ROWS, L, 2 * D_CLN), jnp.float32),
        input_output_aliases={0: 0},
    )(sc_out, cln2)


def kernel(dist_inputs, grid_mask2d, cln, distance_table, region_table):
    dist1 = dist_inputs.reshape(N).astype(jnp.int32)
    mask1 = grid_mask2d.reshape(N).astype(jnp.int32)
    cln2 = cln.reshape(ROWS, L, D_CLN)
    sc_out = _sc_fill(distance_table, region_table, dist1, mask1)
    out = _fill_cln(sc_out.reshape(ROWS, L, 2 * D_CLN), cln2)
    return out.reshape(B, L, L, 2 * D_CLN)


# speedup vs baseline: 1.0318x; 1.0318x over previous
import functools

import jax
import jax.numpy as jnp
from jax import lax
from jax.experimental import pallas as pl
from jax.experimental.pallas import tpu as pltpu
from jax.experimental.pallas import tpu_sc as plsc

B, L, D_CLN, D_EMB = 4, 256, 128, 64
ROWS = B * L
N = ROWS * L
NC, NS = 2, 16
NW = NC * NS
ROWS_PW = ROWS // NW
POS_PW = ROWS_PW * L
GCH = 128
CH = 128
NCH = POS_PW // CH
RB = 64
REP = 1
NBUF = 4


def _sc_body(dt_hbm, rt_hbm, dist_hbm, mask_hbm, out_hbm,
             ct_v, dt_v, rt_v, ctmp_v, dist_v, mask_v, idx_v, stage_v, sem):
    wid = lax.axis_index("s") * NC + lax.axis_index("c")
    base = wid * POS_PW

    sid = lax.axis_index("s")

    @pl.when(sid == 0)
    def _():
        pltpu.sync_copy(dt_hbm, dt_v)
        pltpu.sync_copy(rt_hbm, rt_v)
        for c in range(60):
            for k in range(D_EMB // 16):
                ctmp_v[c, pl.ds(k * 16, 16)] = dt_v[c // 3, pl.ds(k * 16, 16)]
                ctmp_v[c, pl.ds(D_EMB + k * 16, 16)] = (
                    rt_v[c % 3, pl.ds(k * 16, 16)])
        for rep in range(REP):
            pltpu.sync_copy(ctmp_v, ct_v.at[pl.ds(rep * 64, 64)])

    pltpu.sync_copy(dist_hbm.at[pl.ds(base, POS_PW)], dist_v)
    pltpu.sync_copy(mask_hbm.at[pl.ds(base, POS_PW)], mask_v)

    rep_off = jnp.full((16,), lax.rem(sid, REP) * 64, jnp.int32)

    def idx_row(t, _):
        g = wid * ROWS_PW + t
        i = lax.rem(g, L)
        iv = jnp.full((16,), i, jnp.int32)
        for jj in range(L // 16):
            o = t * L + jj * 16
            d = dist_v[pl.ds(o, 16)]
            m = mask_v[pl.ds(o, 16)]
            j = jj * 16 + lax.iota(jnp.int32, 16)
            ge = jnp.where(j >= iv, jnp.int32(1), jnp.int32(0))
            idx_v[pl.ds(o, 16)] = d * 3 + m * (1 + ge) + rep_off
        return _

    lax.fori_loop(0, ROWS_PW, idx_row, None)
    plsc.subcore_barrier()

    LAG = 2
    gdesc = [None] * NBUF
    wdesc = [None] * NBUF

    def fire_write(c):
        b = c % NBUF
        gdesc[b].wait()
        wdesc[b] = pltpu.async_copy(
            stage_v.at[b],
            out_hbm.at[pl.ds(base + c * CH, CH),
                       pl.ds(D_CLN, 2 * D_EMB)], sem.at[NBUF + b])

    for c in range(NCH):
        b = c % NBUF
        if c >= NBUF:
            wdesc[b].wait()
        gdesc[b] = pltpu.async_copy(
            ct_v.at[idx_v.at[pl.ds(c * CH, CH)]],
            stage_v.at[b], sem.at[b])
        if c >= LAG:
            fire_write(c - LAG)
    for c in range(NCH - LAG, NCH):
        fire_write(c)
    for c in range(NCH - NBUF, NCH):
        wdesc[c % NBUF].wait()


@functools.partial(
    pl.kernel,
    out_type=jax.ShapeDtypeStruct((N, 2 * D_CLN), jnp.float32),
    mesh=plsc.VectorSubcoreMesh(core_axis_name="c", subcore_axis_name="s"),
    scratch_types=[
        pltpu.VMEM_SHARED((64 * REP, 2 * D_EMB), jnp.float32),
        pltpu.VMEM((20, D_EMB), jnp.float32),
        pltpu.VMEM((3, D_EMB), jnp.float32),
        pltpu.VMEM((64, 2 * D_EMB), jnp.float32),
        pltpu.VMEM((POS_PW,), jnp.int32),
        pltpu.VMEM((POS_PW,), jnp.int32),
        pltpu.VMEM((POS_PW,), jnp.int32),
        pltpu.VMEM((NBUF, CH, 2 * D_EMB), jnp.float32),
        pltpu.SemaphoreType.DMA((2 * NBUF,)),
    ],
)
def _sc_fill(dt_hbm, rt_hbm, dist_hbm, mask_hbm, out_hbm,
             ct_v, dt_v, rt_v, ctmp_v, dist_v, mask_v, idx_v, stage_v, sem):
    _sc_body(dt_hbm, rt_hbm, dist_hbm, mask_hbm, out_hbm,
             ct_v, dt_v, rt_v, ctmp_v, dist_v, mask_v, idx_v, stage_v, sem)


def _copy_body(prev_ref, cln_ref, out_ref):
    out_ref[...] = cln_ref[...]


def _fill_cln(sc_out, cln2):
    return pl.pallas_call(
        _copy_body,
        grid=(ROWS // RB,),
        in_specs=[
            pl.BlockSpec(memory_space=pltpu.MemorySpace.HBM),
            pl.BlockSpec((RB, L, D_CLN), lambda r: (r, 0, 0)),
        ],
        out_specs=pl.BlockSpec((RB, L, D_CLN), lambda r: (r, 0, 0)),
        out_shape=jax.ShapeDtypeStruct((ROWS, L, 2 * D_CLN), jnp.float32),
        input_output_aliases={0: 0},
    )(sc_out, cln2)


def kernel(dist_inputs, grid_mask2d, cln, distance_table, region_table):
    dist1 = dist_inputs.reshape(N).astype(jnp.int32)
    mask1 = grid_mask2d.reshape(N).astype(jnp.int32)
    cln2 = cln.reshape(ROWS, L, D_CLN)
    sc_out = _sc_fill(distance_table, region_table, dist1, mask1)
    out = _fill_cln(sc_out.reshape(ROWS, L, 2 * D_CLN), cln2)
    return out.reshape(B, L, L, 2 * D_CLN)
